# baseline (speedup 1.0000x reference)
import functools

import jax
import jax.numpy as jnp
from jax import lax
from jax.experimental import pallas as pl
from jax.experimental.pallas import tpu as pltpu
from jax.experimental.pallas import tpu_sc as plsc

N = 10000
E = 320000
H = 128
C_OUT = 64
EPS = 1e-5

NB = 128
NC = 2
NS = 16
NW = NC * NS
T = 80
CPB = 8
NBODY = T // CPB
NCHP = NW * T
EP = NCHP * NB
NP = 10240
ROWS_PER_TILE = NP // NS

_mesh = plsc.VectorSubcoreMesh(core_axis_name="c", subcore_axis_name="s")
_f32 = jnp.float32


@functools.partial(
    pl.kernel,
    out_type=jax.ShapeDtypeStruct((NC * NP, H), _f32),
    mesh=_mesh,
    scratch_types=[
        pltpu.VMEM((CPB, NB), jnp.int32),
        pltpu.VMEM((NB, H), _f32),
        pltpu.VMEM_SHARED((NP, H), _f32),
    ],
)
def _deg_kernel(dst_hbm, zeros_hbm, ones_hbm, out_hbm, d_v, ones_v, acc):
    cid = lax.axis_index("c")
    sid = lax.axis_index("s")
    wid = sid * NC + cid
    r0 = sid * ROWS_PER_TILE
    pltpu.sync_copy(zeros_hbm, ones_v)
    for t in range(ROWS_PER_TILE // NB):
        pltpu.sync_copy(ones_v, acc.at[pl.ds(r0 + t * NB, NB)])
    pltpu.sync_copy(ones_hbm, ones_v)
    plsc.subcore_barrier()

    def body(i, carry):
        row = wid * T + i * CPB
        pltpu.sync_copy(dst_hbm.at[pl.ds(row, CPB)], d_v)
        for j in range(CPB):
            pltpu.sync_copy(ones_v, acc.at[d_v.at[j]], add=True)
        return carry

    lax.fori_loop(0, NBODY, body, 0)
    plsc.subcore_barrier()
    pltpu.sync_copy(acc.at[pl.ds(r0, ROWS_PER_TILE)],
                    out_hbm.at[pl.ds(cid * NP + r0, ROWS_PER_TILE)])


@functools.partial(
    pl.kernel,
    out_type=jax.ShapeDtypeStruct((NC * NP, H), _f32),
    mesh=_mesh,
    scratch_types=[
        pltpu.VMEM((2 * CPB, NB), jnp.int32),
        pltpu.VMEM((NB, H), _f32),
        pltpu.VMEM((NB, H), _f32),
        pltpu.VMEM_SHARED((NP, H), _f32),
        pltpu.SemaphoreType.DMA,
        pltpu.SemaphoreType.DMA,
    ],
)
def _agg_kernel(hp_hbm, sd_hbm, zeros_hbm, out_hbm,
                sd_v, rows0, rows1, acc, sem0, sem1):
    cid = lax.axis_index("c")
    sid = lax.axis_index("s")
    wid = sid * NC + cid
    r0 = sid * ROWS_PER_TILE
    pltpu.sync_copy(zeros_hbm, rows0)
    for t in range(ROWS_PER_TILE // NB):
        pltpu.sync_copy(rows0, acc.at[pl.ds(r0 + t * NB, NB)])
    plsc.subcore_barrier()

    def body(i, carry):
        row = 2 * (wid * T + i * CPB)
        pltpu.sync_copy(sd_hbm.at[pl.ds(row, 2 * CPB)], sd_v)
        for j in range(CPB):
            pltpu.async_copy(hp_hbm.at[sd_v.at[2 * j]], rows0, sem0).wait()
            pltpu.sync_copy(rows0, acc.at[sd_v.at[2 * j + 1]], add=True)
        return carry

    lax.fori_loop(0, NBODY, body, 0)
    plsc.subcore_barrier()
    pltpu.sync_copy(acc.at[pl.ds(r0, ROWS_PER_TILE)],
                    out_hbm.at[pl.ds(cid * NP + r0, ROWS_PER_TILE)])


def _prep_body(degp_ref, x_ref, dis_ref, hp_ref):
    deg = degp_ref[:N, 0:1] + degp_ref[NP:NP + N, 0:1] + 1.0
    dis = lax.rsqrt(deg)
    dis_ref[...] = dis
    hp_ref[...] = x_ref[...] * dis


_prep_tc = pl.pallas_call(
    _prep_body,
    out_shape=[
        jax.ShapeDtypeStruct((N, 1), _f32),
        jax.ShapeDtypeStruct((N, H), _f32),
    ],
)


def _layer_math(S_ref, hp_ref, dis_ref, W_ref, b_ref, g_ref, be_ref):
    dis = dis_ref[...]
    agg = dis * (S_ref[:N] + S_ref[NP:NP + N] + hp_ref[...])
    z = jnp.dot(agg, W_ref[...], preferred_element_type=_f32) + b_ref[...]
    mu = jnp.mean(z, axis=0, keepdims=True)
    d = z - mu
    var = jnp.mean(d * d, axis=0, keepdims=True)
    y = d * lax.rsqrt(var + EPS) * g_ref[...] + be_ref[...]
    return jnp.maximum(y, 0.0), dis


def _layer_body(S_ref, hp_ref, dis_ref, W_ref, b_ref, g_ref, be_ref,
                y_ref, hpn_ref):
    y, dis = _layer_math(S_ref, hp_ref, dis_ref, W_ref, b_ref, g_ref, be_ref)
    y_ref[...] = y
    hpn_ref[...] = y * dis


_layer_tc = pl.pallas_call(
    _layer_body,
    out_shape=[
        jax.ShapeDtypeStruct((N, H), _f32),
        jax.ShapeDtypeStruct((N, H), _f32),
    ],
)


def _final_body(S_ref, hp_ref, dis_ref, W_ref, b_ref, g_ref, be_ref,
                h1_ref, h2_ref, lw1_ref, lb1_ref, lw2_ref, lb2_ref, out_ref):
    h3, _ = _layer_math(S_ref, hp_ref, dis_ref, W_ref, b_ref, g_ref, be_ref)
    m = jnp.maximum(jnp.maximum(h1_ref[...], h2_ref[...]), h3)
    t = jnp.maximum(
        jnp.dot(m, lw1_ref[...], preferred_element_type=_f32) + lb1_ref[...],
        0.0)
    o = jnp.dot(t, lw2_ref[...], preferred_element_type=_f32) + lb2_ref[...]
    mx = jnp.max(o, axis=-1, keepdims=True)
    lse = jnp.log(jnp.sum(jnp.exp(o - mx), axis=-1, keepdims=True)) + mx
    out_ref[...] = o - lse


_final_tc = pl.pallas_call(
    _final_body,
    out_shape=jax.ShapeDtypeStruct((N, C_OUT), _f32),
)


def kernel(x, adj_t, W0, b0, g0, be0, W1, b1, g1, be1, W2, b2, g2, be2,
           lw1, lb1, lw2, lb2):
    src = adj_t[0]
    dst = adj_t[1]
    pad = EP - E
    srcp = jnp.concatenate([src, jnp.zeros((pad,), jnp.int32)])
    dstp = jnp.concatenate(
        [dst, N + (jnp.arange(pad, dtype=jnp.int32) % (NP - N))])
    sd_all = jnp.stack(
        [srcp.reshape(NCHP, NB), dstp.reshape(NCHP, NB)], axis=1
    ).reshape(2 * NCHP, NB)
    d_all = dstp.reshape(NCHP, NB)

    zerosH = jnp.zeros((NB, H), _f32)
    onesH = jnp.ones((NB, H), _f32)

    degp = _deg_kernel(d_all, zerosH, onesH)
    dis, hp0 = _prep_tc(degp, x)

    S = _agg_kernel(hp0, sd_all, zerosH)
    h1, hp1 = _layer_tc(S, hp0, dis, W0, b0.reshape(1, H), g0.reshape(1, H),
                        be0.reshape(1, H))
    S = _agg_kernel(hp1, sd_all, zerosH)
    h2, hp2 = _layer_tc(S, hp1, dis, W1, b1.reshape(1, H), g1.reshape(1, H),
                        be1.reshape(1, H))
    S = _agg_kernel(hp2, sd_all, zerosH)
    out = _final_tc(S, hp2, dis, W2, b2.reshape(1, H), g2.reshape(1, H),
                    be2.reshape(1, H), h1, h2, lw1, lb1.reshape(1, H), lw2,
                    lb2.reshape(1, C_OUT))
    return out


# speedup vs baseline: 1.1108x; 1.1108x over previous
import functools

import jax
import jax.numpy as jnp
from jax import lax
from jax.experimental import pallas as pl
from jax.experimental.pallas import tpu as pltpu
from jax.experimental.pallas import tpu_sc as plsc

N = 10000
E = 320000
H = 128
C_OUT = 64
EPS = 1e-5

NB = 128
NC = 2
NS = 16
NW = NC * NS
T = 80
CPB = 16
NBODY = T // CPB
NCHP = NW * T
EP = NCHP * NB
NP = 10240
ROWS_PER_TILE = NP // NS

_mesh = plsc.VectorSubcoreMesh(core_axis_name="c", subcore_axis_name="s")
_f32 = jnp.float32


@functools.partial(
    pl.kernel,
    out_type=jax.ShapeDtypeStruct((NC * NP, H), _f32),
    mesh=_mesh,
    scratch_types=[
        pltpu.VMEM((CPB, NB), jnp.int32),
        pltpu.VMEM((NB, H), _f32),
        pltpu.VMEM_SHARED((NP, H), _f32),
    ],
)
def _deg_kernel(dst_hbm, zeros_hbm, ones_hbm, out_hbm, d_v, ones_v, acc):
    cid = lax.axis_index("c")
    sid = lax.axis_index("s")
    wid = sid * NC + cid
    r0 = sid * ROWS_PER_TILE
    pltpu.sync_copy(zeros_hbm, ones_v)
    for t in range(ROWS_PER_TILE // NB):
        pltpu.sync_copy(ones_v, acc.at[pl.ds(r0 + t * NB, NB)])
    pltpu.sync_copy(ones_hbm, ones_v)
    plsc.subcore_barrier()

    def body(i, carry):
        row = wid * T + i * CPB
        pltpu.sync_copy(dst_hbm.at[pl.ds(row, CPB)], d_v)
        for j in range(CPB):
            pltpu.sync_copy(ones_v, acc.at[d_v.at[j]], add=True)
        return carry

    lax.fori_loop(0, NBODY, body, 0)
    plsc.subcore_barrier()
    pltpu.sync_copy(acc.at[pl.ds(r0, ROWS_PER_TILE)],
                    out_hbm.at[pl.ds(cid * NP + r0, ROWS_PER_TILE)])


@functools.partial(
    pl.kernel,
    out_type=jax.ShapeDtypeStruct((NC * NP, H), _f32),
    mesh=_mesh,
    scratch_types=[
        pltpu.VMEM((2 * CPB, NB), jnp.int32),
        pltpu.VMEM((NB, H), _f32),
        pltpu.VMEM((NB, H), _f32),
        pltpu.VMEM_SHARED((NP, H), _f32),
        pltpu.SemaphoreType.DMA,
        pltpu.SemaphoreType.DMA,
    ],
)
def _agg_kernel(hp_hbm, sd_hbm, zeros_hbm, out_hbm,
                sd_v, rows0, rows1, acc, sem0, sem1):
    cid = lax.axis_index("c")
    sid = lax.axis_index("s")
    wid = sid * NC + cid
    r0 = sid * ROWS_PER_TILE
    pltpu.sync_copy(zeros_hbm, rows0)
    for t in range(ROWS_PER_TILE // NB):
        pltpu.sync_copy(rows0, acc.at[pl.ds(r0 + t * NB, NB)])
    plsc.subcore_barrier()

    bufs = (rows0, rows1)
    sems = (sem0, sem1)

    def body(i, carry):
        row = 2 * (wid * T + i * CPB)
        pltpu.sync_copy(sd_hbm.at[pl.ds(row, 2 * CPB)], sd_v)
        cps = [pltpu.async_copy(hp_hbm.at[sd_v.at[0]], rows0, sem0), None]
        for j in range(CPB):
            if j + 1 < CPB:
                nb = (j + 1) % 2
                cps[nb] = pltpu.async_copy(
                    hp_hbm.at[sd_v.at[2 * (j + 1)]], bufs[nb], sems[nb])
            cps[j % 2].wait()
            pltpu.sync_copy(bufs[j % 2], acc.at[sd_v.at[2 * j + 1]], add=True)
        return carry

    lax.fori_loop(0, NBODY, body, 0)
    plsc.subcore_barrier()
    pltpu.sync_copy(acc.at[pl.ds(r0, ROWS_PER_TILE)],
                    out_hbm.at[pl.ds(cid * NP + r0, ROWS_PER_TILE)])


def _prep_body(degp_ref, x_ref, dis_ref, hp_ref):
    deg = degp_ref[:N, 0:1] + degp_ref[NP:NP + N, 0:1] + 1.0
    dis = lax.rsqrt(deg)
    dis_ref[...] = dis
    hp_ref[...] = x_ref[...] * dis


_prep_tc = pl.pallas_call(
    _prep_body,
    out_shape=[
        jax.ShapeDtypeStruct((N, 1), _f32),
        jax.ShapeDtypeStruct((N, H), _f32),
    ],
)


def _layer_math(S_ref, hp_ref, dis_ref, W_ref, b_ref, g_ref, be_ref):
    dis = dis_ref[...]
    agg = dis * (S_ref[:N] + S_ref[NP:NP + N] + hp_ref[...])
    z = jnp.dot(agg, W_ref[...], preferred_element_type=_f32) + b_ref[...]
    mu = jnp.mean(z, axis=0, keepdims=True)
    d = z - mu
    var = jnp.mean(d * d, axis=0, keepdims=True)
    y = d * lax.rsqrt(var + EPS) * g_ref[...] + be_ref[...]
    return jnp.maximum(y, 0.0), dis


def _layer_body(S_ref, hp_ref, dis_ref, W_ref, b_ref, g_ref, be_ref,
                y_ref, hpn_ref):
    y, dis = _layer_math(S_ref, hp_ref, dis_ref, W_ref, b_ref, g_ref, be_ref)
    y_ref[...] = y
    hpn_ref[...] = y * dis


_layer_tc = pl.pallas_call(
    _layer_body,
    out_shape=[
        jax.ShapeDtypeStruct((N, H), _f32),
        jax.ShapeDtypeStruct((N, H), _f32),
    ],
)


def _final_body(S_ref, hp_ref, dis_ref, W_ref, b_ref, g_ref, be_ref,
                h1_ref, h2_ref, lw1_ref, lb1_ref, lw2_ref, lb2_ref, out_ref):
    h3, _ = _layer_math(S_ref, hp_ref, dis_ref, W_ref, b_ref, g_ref, be_ref)
    m = jnp.maximum(jnp.maximum(h1_ref[...], h2_ref[...]), h3)
    t = jnp.maximum(
        jnp.dot(m, lw1_ref[...], preferred_element_type=_f32) + lb1_ref[...],
        0.0)
    o = jnp.dot(t, lw2_ref[...], preferred_element_type=_f32) + lb2_ref[...]
    mx = jnp.max(o, axis=-1, keepdims=True)
    lse = jnp.log(jnp.sum(jnp.exp(o - mx), axis=-1, keepdims=True)) + mx
    out_ref[...] = o - lse


_final_tc = pl.pallas_call(
    _final_body,
    out_shape=jax.ShapeDtypeStruct((N, C_OUT), _f32),
)


def kernel(x, adj_t, W0, b0, g0, be0, W1, b1, g1, be1, W2, b2, g2, be2,
           lw1, lb1, lw2, lb2):
    src = adj_t[0]
    dst = adj_t[1]
    pad = EP - E
    srcp = jnp.concatenate([src, jnp.zeros((pad,), jnp.int32)])
    dstp = jnp.concatenate(
        [dst, N + (jnp.arange(pad, dtype=jnp.int32) % (NP - N))])
    sd_all = jnp.stack(
        [srcp.reshape(NCHP, NB), dstp.reshape(NCHP, NB)], axis=1
    ).reshape(2 * NCHP, NB)
    d_all = dstp.reshape(NCHP, NB)

    zerosH = jnp.zeros((NB, H), _f32)
    onesH = jnp.ones((NB, H), _f32)

    degp = _deg_kernel(d_all, zerosH, onesH)
    dis, hp0 = _prep_tc(degp, x)

    S = _agg_kernel(hp0, sd_all, zerosH)
    h1, hp1 = _layer_tc(S, hp0, dis, W0, b0.reshape(1, H), g0.reshape(1, H),
                        be0.reshape(1, H))
    S = _agg_kernel(hp1, sd_all, zerosH)
    h2, hp2 = _layer_tc(S, hp1, dis, W1, b1.reshape(1, H), g1.reshape(1, H),
                        be1.reshape(1, H))
    S = _agg_kernel(hp2, sd_all, zerosH)
    out = _final_tc(S, hp2, dis, W2, b2.reshape(1, H), g2.reshape(1, H),
                    be2.reshape(1, H), h1, h2, lw1, lb1.reshape(1, H), lw2,
                    lb2.reshape(1, C_OUT))
    return out
